# baseline (speedup 1.0000x reference)
import jax
import jax.numpy as jnp
from jax import lax
from jax.experimental import pallas as pl
from jax.experimental.pallas import tpu as pltpu
from jax.experimental.pallas import tpu_sc as plsc

_N = 10000
_D = 256
_H = 128
_E = 160000
_NS = 16

_EPT = _E // _NS
_CH = 80
_NCH = _EPT // _CH


def _seg_body(g, src, dst, out, acc, sidx, didx, rows0, rows1, rows2, sem):
    cid = lax.axis_index("c")
    sid = lax.axis_index("s")
    gt = g.at[cid]

    @pl.when(sid == 0)
    def _():
        pltpu.sync_copy(gt, acc)

    base = pl.multiple_of(sid * _EPT, 8)
    pltpu.sync_copy(src.at[pl.ds(base, _EPT)], sidx)
    pltpu.sync_copy(dst.at[pl.ds(base, _EPT)], didx)
    plsc.subcore_barrier()

    rows = (rows0, rows1, rows2)

    def gather(j, buf):
        off = pl.multiple_of(j * _CH, 8)
        pltpu.async_copy(gt.at[sidx.at[pl.ds(off, _CH)]], buf, sem)

    def scatter(j, buf):
        off = pl.multiple_of(j * _CH, 8)
        pltpu.sync_copy(buf, acc.at[didx.at[pl.ds(off, _CH)]], add=True)

    def wait(buf):
        pltpu.make_async_copy(gt.at[pl.ds(0, _CH)], buf, sem).wait()

    gather(0, rows0)
    gather(1, rows1)

    def body(k, carry):
        for b in range(3):
            j = 3 * k + b
            wait(rows[b])
            gather(j + 2, rows[(b + 2) % 3])
            scatter(j, rows[b])
        return carry

    lax.fori_loop(0, (_NCH - 2) // 3, body, 0)
    wait(rows[(_NCH - 2) % 3])
    scatter(_NCH - 2, rows[(_NCH - 2) % 3])
    wait(rows[(_NCH - 1) % 3])
    scatter(_NCH - 1, rows[(_NCH - 1) % 3])

    plsc.subcore_barrier()

    @pl.when(sid == 0)
    def _():
        pltpu.sync_copy(acc, out.at[cid])


_seg_call = pl.kernel(
    _seg_body,
    out_type=jax.ShapeDtypeStruct((2, _N, _H), jnp.float32),
    mesh=plsc.VectorSubcoreMesh(core_axis_name="c", subcore_axis_name="s"),
    scratch_types=[
        pltpu.VMEM_SHARED((_N, _H), jnp.float32),
        pltpu.VMEM((_EPT,), jnp.int32),
        pltpu.VMEM((_EPT,), jnp.int32),
        pltpu.VMEM((_CH, _H), jnp.float32),
        pltpu.VMEM((_CH, _H), jnp.float32),
        pltpu.VMEM((_CH, _H), jnp.float32),
        pltpu.SemaphoreType.DMA,
    ],
)

_NW = 2 * _NS
_DEG_EPW = _E // _NW
_DEG_FULL = _DEG_EPW // 16
_DEG_TAIL = _DEG_EPW - _DEG_FULL * 16


def _deg_body(dst, out, cnt, didx):
    cid = lax.axis_index("c")
    sid = lax.axis_index("s")
    wid = cid * _NS + sid

    def zero(i, c):
        cnt[pl.ds(i * 16, 16)] = jnp.zeros((16,), jnp.float32)
        return c

    lax.fori_loop(0, _N // 16, zero, 0)

    base = wid * _DEG_EPW
    pltpu.sync_copy(dst.at[pl.ds(base, _DEG_EPW)], didx.at[pl.ds(0, _DEG_EPW)])
    ones = jnp.ones((16,), jnp.float32)

    def body(j, c):
        idx = didx[pl.ds(j * 16, 16)]
        plsc.addupdate_scatter(cnt, [idx], ones)
        return c

    lax.fori_loop(0, _DEG_FULL, body, 0)
    idx = didx[pl.ds(_DEG_FULL * 16, 16)]
    mask = lax.iota(jnp.int32, 16) < _DEG_TAIL
    plsc.addupdate_scatter(cnt, [idx], ones, mask=mask)

    pltpu.sync_copy(cnt, out.at[wid])


_deg_call = pl.kernel(
    _deg_body,
    out_type=jax.ShapeDtypeStruct((_NW, _N), jnp.float32),
    mesh=plsc.VectorSubcoreMesh(core_axis_name="c", subcore_axis_name="s"),
    compiler_params=pltpu.CompilerParams(needs_layout_passes=False),
    scratch_types=[
        pltpu.VMEM((_N,), jnp.float32),
        pltpu.VMEM((_DEG_EPW + 16,), jnp.int32),
    ],
)

_BLK = 400


def _tc1_body(x_ref, w1_ref, degp_ref, g_ref, dinv_ref):
    deg = jnp.sum(degp_ref[...], axis=0)
    dinv = lax.rsqrt(deg + 1.0)
    dinv_ref[...] = dinv
    h = jnp.dot(x_ref[...], w1_ref[...], preferred_element_type=jnp.float32)
    g = h * dinv
    g_ref[0] = g[:, :_H]
    g_ref[1] = g[:, _H:]


_tc1_call = pl.pallas_call(
    _tc1_body,
    grid=(_N // _BLK,),
    in_specs=[pl.BlockSpec((_BLK, _D), lambda i: (i, 0)),
              pl.BlockSpec((_D, _D), lambda i: (0, 0)),
              pl.BlockSpec((_NW, _BLK, 1), lambda i: (0, i, 0))],
    out_specs=[pl.BlockSpec((2, _BLK, _H), lambda i: (0, i, 0)),
               pl.BlockSpec((_BLK, 1), lambda i: (i, 0))],
    out_shape=[jax.ShapeDtypeStruct((2, _N, _H), jnp.float32),
               jax.ShapeDtypeStruct((_N, 1), jnp.float32)],
)


def _tc2_body(t_ref, dinv_ref, b1_ref, w2_ref, x32_ref, g_ref):
    t = jnp.concatenate([t_ref[0], t_ref[1]], axis=1)
    dinv = dinv_ref[...]
    x32 = jnp.maximum(dinv * t + b1_ref[...], 0.0)
    x32_ref[...] = x32
    g = jnp.dot(x32, w2_ref[...], preferred_element_type=jnp.float32) * dinv
    g_ref[0] = g[:, :_H]
    g_ref[1] = g[:, _H:]


def _tc3_body(t_ref, dinv_ref, b2_ref, x_ref, lw_ref, lb_ref, out_ref):
    t = jnp.concatenate([t_ref[0], t_ref[1]], axis=1)
    skip = jnp.dot(x_ref[...], lw_ref[...], preferred_element_type=jnp.float32)
    out_ref[...] = dinv_ref[...] * t + b2_ref[...] + skip + lb_ref[...]


def _row_spec(w):
    return pl.BlockSpec((_BLK, w), lambda i: (i, 0))


def _full_spec(shape):
    return pl.BlockSpec(shape, lambda i: tuple(0 for _ in shape))


def _stk_spec():
    return pl.BlockSpec((2, _BLK, _H), lambda i: (0, i, 0))


_GRID = _N // _BLK

_tc2_call = pl.pallas_call(
    _tc2_body,
    grid=(_GRID,),
    in_specs=[_stk_spec(), _row_spec(1),
              _full_spec((1, _D)), _full_spec((_D, _D))],
    out_specs=[_row_spec(_D), _stk_spec()],
    out_shape=[
        jax.ShapeDtypeStruct((_N, _D), jnp.float32),
        jax.ShapeDtypeStruct((2, _N, _H), jnp.float32),
    ],
)

_tc3_call = pl.pallas_call(
    _tc3_body,
    grid=(_GRID,),
    in_specs=[_stk_spec(), _row_spec(1),
              _full_spec((1, _D)), _row_spec(_D), _full_spec((_D, _D)),
              _full_spec((1, _D))],
    out_specs=[_row_spec(_D)],
    out_shape=[jax.ShapeDtypeStruct((_N, _D), jnp.float32)],
)


def kernel(x, edge_index, W1, b1, W2, b2, lin_W, lin_b):
    src = edge_index[0].astype(jnp.int32)
    dst = edge_index[1].astype(jnp.int32)
    b1r = b1.reshape(1, _D)
    b2r = b2.reshape(1, _D)
    lbr = lin_b.reshape(1, _D)

    degp = _deg_call(dst).reshape(_NW, _N, 1)
    g1, dinv = _tc1_call(x, W1, degp)
    t1 = _seg_call(g1, src, dst)
    x32, g2 = _tc2_call(t1, dinv, b1r, W2)
    t2 = _seg_call(g2, src, dst)
    (out,) = _tc3_call(t2, dinv, b2r, x, lin_W, lbr)
    return (x32, out)


# speedup vs baseline: 1.3488x; 1.3488x over previous
import jax
import jax.numpy as jnp
from jax import lax
from jax.experimental import pallas as pl
from jax.experimental.pallas import tpu as pltpu
from jax.experimental.pallas import tpu_sc as plsc

_N = 10000
_D = 256
_H = 128
_E = 160000
_NS = 16

_EPT = _E // _NS
_CH = 80
_NCH = _EPT // _CH


def _seg_body(g, src, dst, out, acc, sidx, didx, rows0, rows1, rows2, sem):
    cid = lax.axis_index("c")
    sid = lax.axis_index("s")
    gt = g.at[cid]

    @pl.when(sid == 0)
    def _():
        pltpu.sync_copy(gt, acc)

    base = pl.multiple_of(sid * _EPT, 8)
    pltpu.sync_copy(src.at[pl.ds(base, _EPT)], sidx)
    pltpu.sync_copy(dst.at[pl.ds(base, _EPT)], didx)
    plsc.subcore_barrier()

    rows = (rows0, rows1, rows2)

    def gather(j, buf):
        off = pl.multiple_of(j * _CH, 8)
        pltpu.async_copy(gt.at[sidx.at[pl.ds(off, _CH)]], buf, sem)

    def scatter(j, buf):
        off = pl.multiple_of(j * _CH, 8)
        pltpu.sync_copy(buf, acc.at[didx.at[pl.ds(off, _CH)]], add=True)

    def wait(buf):
        pltpu.make_async_copy(gt.at[pl.ds(0, _CH)], buf, sem).wait()

    gather(0, rows0)
    gather(1, rows1)

    def body(k, carry):
        for b in range(3):
            j = 3 * k + b
            wait(rows[b])
            gather(j + 2, rows[(b + 2) % 3])
            scatter(j, rows[b])
        return carry

    lax.fori_loop(0, (_NCH - 2) // 3, body, 0)
    wait(rows[(_NCH - 2) % 3])
    scatter(_NCH - 2, rows[(_NCH - 2) % 3])
    wait(rows[(_NCH - 1) % 3])
    scatter(_NCH - 1, rows[(_NCH - 1) % 3])

    plsc.subcore_barrier()

    @pl.when(sid == 0)
    def _():
        pltpu.sync_copy(acc, out.at[cid])


_seg_call = pl.kernel(
    _seg_body,
    out_type=jax.ShapeDtypeStruct((2, _N, _H), jnp.float32),
    mesh=plsc.VectorSubcoreMesh(core_axis_name="c", subcore_axis_name="s"),
    scratch_types=[
        pltpu.VMEM_SHARED((_N, _H), jnp.float32),
        pltpu.VMEM((_EPT,), jnp.int32),
        pltpu.VMEM((_EPT,), jnp.int32),
        pltpu.VMEM((_CH, _H), jnp.float32),
        pltpu.VMEM((_CH, _H), jnp.float32),
        pltpu.VMEM((_CH, _H), jnp.float32),
        pltpu.SemaphoreType.DMA,
    ],
)

_NW = 2 * _NS
_DEG_EPW = _E // _NW
_DEG_FULL = _DEG_EPW // 16
_DEG_TAIL = _DEG_EPW - _DEG_FULL * 16


def _deg_body(dst, out, cnt, didx):
    cid = lax.axis_index("c")
    sid = lax.axis_index("s")
    wid = cid * _NS + sid

    def zero(i, c):
        cnt[pl.ds(i * 16, 16)] = jnp.zeros((16,), jnp.float32)
        return c

    lax.fori_loop(0, _N // 16, zero, 0)

    base = wid * _DEG_EPW
    pltpu.sync_copy(dst.at[pl.ds(base, _DEG_EPW)], didx.at[pl.ds(0, _DEG_EPW)])
    ones = jnp.ones((16,), jnp.float32)

    def body(j, c):
        idx = didx[pl.ds(j * 16, 16)]
        plsc.addupdate_scatter(cnt, [idx], ones)
        return c

    lax.fori_loop(0, _DEG_FULL, body, 0)
    idx = didx[pl.ds(_DEG_FULL * 16, 16)]
    mask = lax.iota(jnp.int32, 16) < _DEG_TAIL
    plsc.addupdate_scatter(cnt, [idx], ones, mask=mask)

    pltpu.sync_copy(cnt, out.at[wid])


_deg_call = pl.kernel(
    _deg_body,
    out_type=jax.ShapeDtypeStruct((_NW, _N), jnp.float32),
    mesh=plsc.VectorSubcoreMesh(core_axis_name="c", subcore_axis_name="s"),
    compiler_params=pltpu.CompilerParams(needs_layout_passes=False),
    scratch_types=[
        pltpu.VMEM((_N,), jnp.float32),
        pltpu.VMEM((_DEG_EPW + 16,), jnp.int32),
    ],
)

_BLK = 400


def _dinv_body(deg_ref, dinv_ref):
    deg = jnp.sum(deg_ref[...], axis=0)
    dinv_ref[...] = lax.rsqrt(deg + 1.0)[:, None]


_dinv_call = pl.pallas_call(
    _dinv_body,
    out_shape=jax.ShapeDtypeStruct((_N, 1), jnp.float32),
)


def _tc1_body(x_ref, w1_ref, dinv_ref, g_ref):
    h = jnp.dot(x_ref[...], w1_ref[...], preferred_element_type=jnp.float32)
    g = h * dinv_ref[...]
    g_ref[0] = g[:, :_H]
    g_ref[1] = g[:, _H:]


_tc1_call = pl.pallas_call(
    _tc1_body,
    grid=(_N // _BLK,),
    in_specs=[pl.BlockSpec((_BLK, _D), lambda i: (i, 0)),
              pl.BlockSpec((_D, _D), lambda i: (0, 0)),
              pl.BlockSpec((_BLK, 1), lambda i: (i, 0))],
    out_specs=[pl.BlockSpec((2, _BLK, _H), lambda i: (0, i, 0))],
    out_shape=[jax.ShapeDtypeStruct((2, _N, _H), jnp.float32)],
)


def _tc2_body(t_ref, dinv_ref, b1_ref, w2_ref, x32_ref, g_ref):
    t = jnp.concatenate([t_ref[0], t_ref[1]], axis=1)
    dinv = dinv_ref[...]
    x32 = jnp.maximum(dinv * t + b1_ref[...], 0.0)
    x32_ref[...] = x32
    g = jnp.dot(x32, w2_ref[...], preferred_element_type=jnp.float32) * dinv
    g_ref[0] = g[:, :_H]
    g_ref[1] = g[:, _H:]


def _tc3_body(t_ref, dinv_ref, b2_ref, x_ref, lw_ref, lb_ref, out_ref):
    t = jnp.concatenate([t_ref[0], t_ref[1]], axis=1)
    skip = jnp.dot(x_ref[...], lw_ref[...], preferred_element_type=jnp.float32)
    out_ref[...] = dinv_ref[...] * t + b2_ref[...] + skip + lb_ref[...]


def _row_spec(w):
    return pl.BlockSpec((_BLK, w), lambda i: (i, 0))


def _full_spec(shape):
    return pl.BlockSpec(shape, lambda i: tuple(0 for _ in shape))


def _stk_spec():
    return pl.BlockSpec((2, _BLK, _H), lambda i: (0, i, 0))


_GRID = _N // _BLK

_tc2_call = pl.pallas_call(
    _tc2_body,
    grid=(_GRID,),
    in_specs=[_stk_spec(), _row_spec(1),
              _full_spec((1, _D)), _full_spec((_D, _D))],
    out_specs=[_row_spec(_D), _stk_spec()],
    out_shape=[
        jax.ShapeDtypeStruct((_N, _D), jnp.float32),
        jax.ShapeDtypeStruct((2, _N, _H), jnp.float32),
    ],
)

_tc3_call = pl.pallas_call(
    _tc3_body,
    grid=(_GRID,),
    in_specs=[_stk_spec(), _row_spec(1),
              _full_spec((1, _D)), _row_spec(_D), _full_spec((_D, _D)),
              _full_spec((1, _D))],
    out_specs=[_row_spec(_D)],
    out_shape=[jax.ShapeDtypeStruct((_N, _D), jnp.float32)],
)


def kernel(x, edge_index, W1, b1, W2, b2, lin_W, lin_b):
    src = edge_index[0].astype(jnp.int32)
    dst = edge_index[1].astype(jnp.int32)
    b1r = b1.reshape(1, _D)
    b2r = b2.reshape(1, _D)
    lbr = lin_b.reshape(1, _D)

    deg = _deg_call(dst)
    dinv = _dinv_call(deg)
    (g1,) = _tc1_call(x, W1, dinv)
    t1 = _seg_call(g1, src, dst)
    x32, g2 = _tc2_call(t1, dinv, b1r, W2)
    t2 = _seg_call(g2, src, dst)
    (out,) = _tc3_call(t2, dinv, b2r, x, lin_W, lbr)
    return (x32, out)


# speedup vs baseline: 1.4080x; 1.0439x over previous
import jax
import jax.numpy as jnp
from jax import lax
from jax.experimental import pallas as pl
from jax.experimental.pallas import tpu as pltpu
from jax.experimental.pallas import tpu_sc as plsc

_N = 10000
_D = 256
_H = 128
_E = 160000
_NS = 16

_EPT = _E // _NS
_CH = 40
_NCH = _EPT // _CH
_NBUF = 6


def _seg_body(g, src, dst, out, acc, sidx, didx, *rest):
    rows = rest[:_NBUF]
    sem = rest[_NBUF]
    cid = lax.axis_index("c")
    sid = lax.axis_index("s")
    gt = g.at[cid]

    @pl.when(sid == 0)
    def _():
        pltpu.sync_copy(gt, acc)

    base = pl.multiple_of(sid * _EPT, 8)
    pltpu.sync_copy(src.at[pl.ds(base, _EPT)], sidx)
    pltpu.sync_copy(dst.at[pl.ds(base, _EPT)], didx)
    plsc.subcore_barrier()

    def gather(j, buf):
        off = pl.multiple_of(j * _CH, 8)
        pltpu.async_copy(gt.at[sidx.at[pl.ds(off, _CH)]], buf, sem)

    def scatter(j, buf):
        off = pl.multiple_of(j * _CH, 8)
        pltpu.sync_copy(buf, acc.at[didx.at[pl.ds(off, _CH)]], add=True)

    def wait(buf):
        pltpu.make_async_copy(gt.at[pl.ds(0, _CH)], buf, sem).wait()

    for j in range(_NBUF - 1):
        gather(j, rows[j])

    _NMAIN = (_NCH - (_NBUF - 1)) // _NBUF

    def body(k, carry):
        for b in range(_NBUF):
            j = _NBUF * k + b
            wait(rows[b])
            gather(j + _NBUF - 1, rows[(b + _NBUF - 1) % _NBUF])
            scatter(j, rows[b])
        return carry

    lax.fori_loop(0, _NMAIN, body, 0)
    for j in range(_NMAIN * _NBUF, _NCH):
        wait(rows[j % _NBUF])
        if j + _NBUF - 1 < _NCH:
            gather(j + _NBUF - 1, rows[(j + _NBUF - 1) % _NBUF])
        scatter(j, rows[j % _NBUF])

    plsc.subcore_barrier()

    @pl.when(sid == 0)
    def _():
        pltpu.sync_copy(acc, out.at[cid])


_seg_call = pl.kernel(
    _seg_body,
    out_type=jax.ShapeDtypeStruct((2, _N, _H), jnp.float32),
    mesh=plsc.VectorSubcoreMesh(core_axis_name="c", subcore_axis_name="s"),
    scratch_types=[
        pltpu.VMEM_SHARED((_N, _H), jnp.float32),
        pltpu.VMEM((_EPT,), jnp.int32),
        pltpu.VMEM((_EPT,), jnp.int32),
    ] + [pltpu.VMEM((_CH, _H), jnp.float32) for _ in range(_NBUF)] + [
        pltpu.SemaphoreType.DMA,
    ],
)

_NW = 2 * _NS
_DEG_EPW = _E // _NW
_DEG_FULL = _DEG_EPW // 16
_DEG_TAIL = _DEG_EPW - _DEG_FULL * 16


def _deg_body(dst, out, cnt, didx):
    cid = lax.axis_index("c")
    sid = lax.axis_index("s")
    wid = cid * _NS + sid

    def zero(i, c):
        cnt[pl.ds(i * 16, 16)] = jnp.zeros((16,), jnp.float32)
        return c

    lax.fori_loop(0, _N // 16, zero, 0)

    base = wid * _DEG_EPW
    pltpu.sync_copy(dst.at[pl.ds(base, _DEG_EPW)], didx.at[pl.ds(0, _DEG_EPW)])
    ones = jnp.ones((16,), jnp.float32)

    def body(j, c):
        idx = didx[pl.ds(j * 16, 16)]
        plsc.addupdate_scatter(cnt, [idx], ones)
        return c

    lax.fori_loop(0, _DEG_FULL, body, 0)
    idx = didx[pl.ds(_DEG_FULL * 16, 16)]
    mask = lax.iota(jnp.int32, 16) < _DEG_TAIL
    plsc.addupdate_scatter(cnt, [idx], ones, mask=mask)

    pltpu.sync_copy(cnt, out.at[wid])


_deg_call = pl.kernel(
    _deg_body,
    out_type=jax.ShapeDtypeStruct((_NW, _N), jnp.float32),
    mesh=plsc.VectorSubcoreMesh(core_axis_name="c", subcore_axis_name="s"),
    compiler_params=pltpu.CompilerParams(needs_layout_passes=False),
    scratch_types=[
        pltpu.VMEM((_N,), jnp.float32),
        pltpu.VMEM((_DEG_EPW + 16,), jnp.int32),
    ],
)

_BLK = 400


def _dinv_body(deg_ref, dinv_ref):
    deg = jnp.sum(deg_ref[...], axis=0)
    dinv_ref[...] = lax.rsqrt(deg + 1.0)[:, None]


_dinv_call = pl.pallas_call(
    _dinv_body,
    out_shape=jax.ShapeDtypeStruct((_N, 1), jnp.float32),
)


def _tc1_body(x_ref, w1_ref, dinv_ref, g_ref):
    h = jnp.dot(x_ref[...], w1_ref[...], preferred_element_type=jnp.float32)
    g = h * dinv_ref[...]
    g_ref[0] = g[:, :_H]
    g_ref[1] = g[:, _H:]


_tc1_call = pl.pallas_call(
    _tc1_body,
    grid=(_N // _BLK,),
    in_specs=[pl.BlockSpec((_BLK, _D), lambda i: (i, 0)),
              pl.BlockSpec((_D, _D), lambda i: (0, 0)),
              pl.BlockSpec((_BLK, 1), lambda i: (i, 0))],
    out_specs=[pl.BlockSpec((2, _BLK, _H), lambda i: (0, i, 0))],
    out_shape=[jax.ShapeDtypeStruct((2, _N, _H), jnp.float32)],
)


def _tc2_body(t_ref, dinv_ref, b1_ref, w2_ref, x32_ref, g_ref):
    t = jnp.concatenate([t_ref[0], t_ref[1]], axis=1)
    dinv = dinv_ref[...]
    x32 = jnp.maximum(dinv * t + b1_ref[...], 0.0)
    x32_ref[...] = x32
    g = jnp.dot(x32, w2_ref[...], preferred_element_type=jnp.float32) * dinv
    g_ref[0] = g[:, :_H]
    g_ref[1] = g[:, _H:]


def _tc3_body(t_ref, dinv_ref, b2_ref, x_ref, lw_ref, lb_ref, out_ref):
    t = jnp.concatenate([t_ref[0], t_ref[1]], axis=1)
    skip = jnp.dot(x_ref[...], lw_ref[...], preferred_element_type=jnp.float32)
    out_ref[...] = dinv_ref[...] * t + b2_ref[...] + skip + lb_ref[...]


def _row_spec(w):
    return pl.BlockSpec((_BLK, w), lambda i: (i, 0))


def _full_spec(shape):
    return pl.BlockSpec(shape, lambda i: tuple(0 for _ in shape))


def _stk_spec():
    return pl.BlockSpec((2, _BLK, _H), lambda i: (0, i, 0))


_GRID = _N // _BLK

_tc2_call = pl.pallas_call(
    _tc2_body,
    grid=(_GRID,),
    in_specs=[_stk_spec(), _row_spec(1),
              _full_spec((1, _D)), _full_spec((_D, _D))],
    out_specs=[_row_spec(_D), _stk_spec()],
    out_shape=[
        jax.ShapeDtypeStruct((_N, _D), jnp.float32),
        jax.ShapeDtypeStruct((2, _N, _H), jnp.float32),
    ],
)

_tc3_call = pl.pallas_call(
    _tc3_body,
    grid=(_GRID,),
    in_specs=[_stk_spec(), _row_spec(1),
              _full_spec((1, _D)), _row_spec(_D), _full_spec((_D, _D)),
              _full_spec((1, _D))],
    out_specs=[_row_spec(_D)],
    out_shape=[jax.ShapeDtypeStruct((_N, _D), jnp.float32)],
)


def kernel(x, edge_index, W1, b1, W2, b2, lin_W, lin_b):
    src = edge_index[0].astype(jnp.int32)
    dst = edge_index[1].astype(jnp.int32)
    b1r = b1.reshape(1, _D)
    b2r = b2.reshape(1, _D)
    lbr = lin_b.reshape(1, _D)

    deg = _deg_call(dst)
    dinv = _dinv_call(deg)
    (g1,) = _tc1_call(x, W1, dinv)
    t1 = _seg_call(g1, src, dst)
    x32, g2 = _tc2_call(t1, dinv, b1r, W2)
    t2 = _seg_call(g2, src, dst)
    (out,) = _tc3_call(t2, dinv, b2r, x, lin_W, lbr)
    return (x32, out)
